# baseline (speedup 1.0000x reference)
import jax
import jax.numpy as jnp
from jax.experimental import pallas as pl
from jax.experimental.pallas import tpu as pltpu


def _shift_max(y, s, axis):
    pad_shape = list(y.shape)
    pad_shape[axis] = s
    pad = jnp.full(pad_shape, -jnp.inf, y.dtype)
    idx = [slice(None)] * y.ndim
    idx[axis] = slice(None, y.shape[axis] - s)
    return jnp.maximum(y, jnp.concatenate([pad, y[tuple(idx)]], axis=axis))


def _cummax_2level(y):
    h, w = y.shape
    nt = h // 8
    z = y.reshape(nt, 8, w)
    for s in (1, 2, 4):
        z = _shift_max(z, s, axis=1)
    m = z[:, 7]
    s = 1
    while s < nt:
        m = _shift_max(m, s, axis=0)
        s *= 2
    p = jnp.concatenate(
        [jnp.full((1, w), -jnp.inf, y.dtype), m[:-1]], axis=0
    )
    z = jnp.maximum(z, p[:, None, :])
    return z.reshape(h, w)


def _cummax_body(x_ref, o_ref):
    nb = x_ref.shape[0]
    for ib in range(nb):
        o_ref[ib, 0] = _cummax_2level(x_ref[ib, 0])


def kernel(x):
    b, c, h, w = x.shape
    nb = 2 if b % 2 == 0 else 1
    return pl.pallas_call(
        _cummax_body,
        grid=(b // nb,),
        in_specs=[pl.BlockSpec((nb, c, h, w), lambda i: (i, 0, 0, 0))],
        out_specs=pl.BlockSpec((nb, c, h, w), lambda i: (i, 0, 0, 0)),
        out_shape=jax.ShapeDtypeStruct(x.shape, x.dtype),
        compiler_params=pltpu.CompilerParams(
            dimension_semantics=("parallel",),
        ),
    )(x)


# speedup vs baseline: 1.8336x; 1.8336x over previous
import jax
import jax.numpy as jnp
from jax.experimental import pallas as pl
from jax.experimental.pallas import tpu as pltpu


def _cummax_body(x_ref, o_ref):
    nb, _, h, w = x_ref.shape
    rows = jax.lax.broadcasted_iota(jnp.int32, (8, 1), 0)
    neg_inf = jnp.float32(-jnp.inf)
    for ib in range(nb):
        def tile_step(v, carry):
            t = x_ref[ib, 0, pl.ds(v * 8, 8), :]
            t3 = t
            for s in (1, 2, 4):
                r = pltpu.roll(t3, s, axis=0)
                t3 = jnp.maximum(t3, jnp.where(rows >= s, r, neg_inf))
            tot = t
            for s in (1, 2, 4):
                tot = jnp.maximum(tot, pltpu.roll(tot, s, axis=0))
            o_ref[ib, 0, pl.ds(v * 8, 8), :] = jnp.maximum(t3, carry)
            return jnp.maximum(carry, tot)

        carry0 = jnp.full((8, w), neg_inf, jnp.float32)
        jax.lax.fori_loop(0, h // 8, tile_step, carry0, unroll=4)


def kernel(x):
    b, c, h, w = x.shape
    nb = 2 if b % 2 == 0 else 1
    return pl.pallas_call(
        _cummax_body,
        grid=(b // nb,),
        in_specs=[pl.BlockSpec((nb, c, h, w), lambda i: (i, 0, 0, 0))],
        out_specs=pl.BlockSpec((nb, c, h, w), lambda i: (i, 0, 0, 0)),
        out_shape=jax.ShapeDtypeStruct(x.shape, x.dtype),
        compiler_params=pltpu.CompilerParams(
            dimension_semantics=("parallel",),
        ),
    )(x)
